# Initial kernel scaffold; baseline (speedup 1.0000x reference)
import jax
import jax.numpy as jnp
from jax.experimental import pallas as pl


def kernel(input, W):
    raise NotImplementedError("write your pallas kernel here")


# speedup vs baseline: 1.3067x; 1.3067x over previous
import functools

import jax
import jax.numpy as jnp
from jax import lax
from jax.experimental import pallas as pl
from jax.experimental.pallas import tpu as pltpu
from jax.experimental.pallas import tpu_sc as plsc

BATCH = 4096
HIST_LEN = 200
EMB = 32

_NC = 2
_NS = 16
_NW = _NC * _NS

_N = BATCH * HIST_LEN
_PER_W = _N // _NW
_CHUNK = 128
_NCHUNK = _PER_W // _CHUNK


def _gather_kernel(idx_hbm, table_hbm, out_hbm, idx_v, rows_v, sem):
    wid = lax.axis_index("s") * _NC + lax.axis_index("c")
    pltpu.sync_copy(idx_hbm.at[wid], idx_v)
    base = wid * _PER_W

    def step(g, carry):
        pltpu.async_copy(table_hbm.at[idx_v.at[g]], rows_v, sem).wait()
        pltpu.sync_copy(rows_v, out_hbm.at[pl.ds(base + g * _CHUNK, _CHUNK)])
        return carry

    lax.fori_loop(0, _NCHUNK, step, 0)


@jax.jit
def _gather(idx, table):
    run = pl.kernel(
        _gather_kernel,
        out_type=jax.ShapeDtypeStruct((_N, EMB), jnp.float32),
        mesh=plsc.VectorSubcoreMesh(core_axis_name="c", subcore_axis_name="s"),
        scratch_types=[
            pltpu.VMEM((_NCHUNK, _CHUNK), jnp.int32),
            pltpu.VMEM((_CHUNK, EMB), jnp.float32),
            pltpu.SemaphoreType.DMA,
        ],
        compiler_params=pltpu.CompilerParams(use_tc_tiling_on_sc=False),
    )
    return run(idx, table)


def kernel(input, W):
    idx = input.reshape(_NW, _NCHUNK, _CHUNK).astype(jnp.int32)
    out = _gather(idx, W)
    return out.reshape(BATCH, HIST_LEN, EMB, 1)


# speedup vs baseline: 1.4961x; 1.1450x over previous
import functools

import jax
import jax.numpy as jnp
from jax import lax
from jax.experimental import pallas as pl
from jax.experimental.pallas import tpu as pltpu
from jax.experimental.pallas import tpu_sc as plsc

BATCH = 4096
HIST_LEN = 200
EMB = 32

_NC = 2
_NS = 16
_NW = _NC * _NS

_N = BATCH * HIST_LEN
_PER_W = _N // _NW
_CHUNK = 128
_NCHUNK = _PER_W // _CHUNK
_K = 4
_BLK = _K * _CHUNK
_NBLK = _NCHUNK // _K
_NPAIR = _NBLK // 2


def _gather_kernel(idx_hbm, table_hbm, out_hbm,
                   idx_v, rows_a, rows_b, gsem_a, gsem_b, osem_a, osem_b):
    wid = lax.axis_index("s") * _NC + lax.axis_index("c")
    pltpu.sync_copy(idx_hbm.at[wid], idx_v)
    base = wid * _PER_W

    def fire(rows, gsem, blk):
        for j in range(_K):
            pltpu.async_copy(
                table_hbm.at[idx_v.at[blk * _K + j]],
                rows.at[pl.ds(j * _CHUNK, _CHUNK)],
                gsem,
            )

    def drain(rows, gsem):
        pltpu.make_async_copy(out_hbm.at[pl.ds(0, _BLK)], rows, gsem).wait()

    fire(rows_a, gsem_a, 0)
    fire(rows_b, gsem_b, 1)

    def body(p, carry):
        blk_a = 2 * p
        blk_b = 2 * p + 1
        drain(rows_a, gsem_a)
        out_a = pltpu.async_copy(
            rows_a, out_hbm.at[pl.ds(base + blk_a * _BLK, _BLK)], osem_a)
        out_a.wait()
        fire(rows_a, gsem_a, lax.rem(blk_a + 2, _NBLK))
        drain(rows_b, gsem_b)
        out_b = pltpu.async_copy(
            rows_b, out_hbm.at[pl.ds(base + blk_b * _BLK, _BLK)], osem_b)
        out_b.wait()
        fire(rows_b, gsem_b, lax.rem(blk_b + 2, _NBLK))
        return carry

    lax.fori_loop(0, _NPAIR, body, 0)
    drain(rows_a, gsem_a)
    drain(rows_b, gsem_b)


@jax.jit
def _gather(idx, table):
    run = pl.kernel(
        _gather_kernel,
        out_type=jax.ShapeDtypeStruct((_N, EMB), jnp.float32),
        mesh=plsc.VectorSubcoreMesh(core_axis_name="c", subcore_axis_name="s"),
        scratch_types=[
            pltpu.VMEM((_NCHUNK, _CHUNK), jnp.int32),
            pltpu.VMEM((_BLK, EMB), jnp.float32),
            pltpu.VMEM((_BLK, EMB), jnp.float32),
            pltpu.SemaphoreType.DMA,
            pltpu.SemaphoreType.DMA,
            pltpu.SemaphoreType.DMA,
            pltpu.SemaphoreType.DMA,
        ],
        compiler_params=pltpu.CompilerParams(use_tc_tiling_on_sc=False),
    )
    return run(idx, table)


def kernel(input, W):
    idx = input.reshape(_NW, _NCHUNK, _CHUNK).astype(jnp.int32)
    out = _gather(idx, W)
    return out.reshape(BATCH, HIST_LEN, EMB, 1)
